# baseline (speedup 1.0000x reference)
import functools

import jax
import jax.numpy as jnp
from jax import lax
from jax.experimental import pallas as pl
from jax.experimental.pallas import tpu as pltpu
from jax.experimental.pallas import tpu_sc as plsc

N = 10000
D = 128
H = 128
OUT = 16
G = 64

NC = 2
NS = 16
NW = NC * NS
CHUNK = 128
CPW = 79
EPAD = NW * CPW * CHUNK
NPAD = 10240
RPT = NPAD // NS
RCH = RPT // CHUNK

_f32 = jnp.float32


def _sc_mesh():
  return plsc.VectorSubcoreMesh(
      core_axis_name="c", subcore_axis_name="s",
      num_cores=NC, num_subcores=NS)


@functools.cache
def _sc_agg_kernel():
  @functools.partial(
      pl.kernel,
      out_type=jax.ShapeDtypeStruct((NC * NPAD, H), _f32),
      mesh=_sc_mesh(),
      scratch_types=[
          pltpu.VMEM_SHARED((NPAD, H), _f32),
          pltpu.VMEM((CPW, CHUNK), jnp.int32),
          pltpu.VMEM((CPW, CHUNK), jnp.int32),
          pltpu.VMEM((CHUNK, H), _f32),
          pltpu.SemaphoreType.DMA,
      ],
  )
  def body(h_hbm, src_hbm, dst_hbm, z_hbm, out_hbm,
           acc, src_v, dst_v, rows_v, sem):
    cid = lax.axis_index("c")
    sid = lax.axis_index("s")
    wid = sid * NC + cid
    pltpu.sync_copy(z_hbm, rows_v)
    for j in range(RCH):
      pltpu.sync_copy(rows_v, acc.at[pl.ds(sid * RPT + j * CHUNK, CHUNK)])
    pltpu.sync_copy(src_hbm.at[wid], src_v)
    pltpu.sync_copy(dst_hbm.at[wid], dst_v)
    plsc.subcore_barrier()

    def step(c, carry):
      pltpu.async_copy(h_hbm.at[src_v.at[c]], rows_v, sem).wait()
      pltpu.sync_copy(rows_v, acc.at[dst_v.at[c]], add=True)
      return carry

    lax.fori_loop(0, CPW, step, 0, unroll=False)
    plsc.subcore_barrier()
    for j in range(RCH):
      r0 = sid * RPT + j * CHUNK
      pltpu.sync_copy(acc.at[pl.ds(r0, CHUNK)], rows_v)
      pltpu.sync_copy(rows_v, out_hbm.at[pl.ds(cid * NPAD + r0, CHUNK)])

  return body


def _sc_agg(h, src_p, dst_p, z128):
  return _sc_agg_kernel()(h, src_p, dst_p, z128)


_BR = 2000


def _mlp_body(x_ref, w1_ref, b1_ref, w2_ref, b2_ref, o_ref):
  x = x_ref[...]
  t = jnp.maximum(
      jnp.dot(x, w1_ref[...], preferred_element_type=_f32) + b1_ref[...], 0.0)
  o_ref[...] = jnp.dot(t, w2_ref[...], preferred_element_type=_f32) + b2_ref[...]


def _mlp(nodes, w1, b1, w2, b2):
  grid = N // _BR
  return pl.pallas_call(
      _mlp_body,
      grid=(grid,),
      in_specs=[
          pl.BlockSpec((_BR, D), lambda i: (i, 0)),
          pl.BlockSpec((D, H), lambda i: (0, 0)),
          pl.BlockSpec((1, H), lambda i: (0, 0)),
          pl.BlockSpec((H, H), lambda i: (0, 0)),
          pl.BlockSpec((1, H), lambda i: (0, 0)),
      ],
      out_specs=pl.BlockSpec((_BR, H), lambda i: (i, 0)),
      out_shape=jax.ShapeDtypeStruct((N, H), _f32),
  )(nodes, w1, b1, w2, b2)


def _layer_body(h_ref, aa_ref, ab_ref, da_ref, db_ref,
                ws_ref, wn_ref, b_ref, g_ref, be_ref, o_ref):
  deg = jnp.maximum(da_ref[...] + db_ref[...], 1.0)
  agg = (aa_ref[...] + ab_ref[...]) / deg
  r = (jnp.dot(h_ref[...], ws_ref[...], preferred_element_type=_f32)
       + jnp.dot(agg, wn_ref[...], preferred_element_type=_f32)
       + b_ref[...])
  r = jnp.maximum(r, 0.0)
  mu = jnp.mean(r, axis=-1, keepdims=True)
  var = jnp.mean((r - mu) ** 2, axis=-1, keepdims=True)
  o_ref[...] = (r - mu) * lax.rsqrt(var + 1e-5) * g_ref[...] + be_ref[...]


def _layer(h, agg_a, agg_b, deg_a, deg_b, ws, wn, b, g, be):
  grid = N // _BR
  return pl.pallas_call(
      _layer_body,
      grid=(grid,),
      in_specs=[
          pl.BlockSpec((_BR, H), lambda i: (i, 0)),
          pl.BlockSpec((_BR, H), lambda i: (i, 0)),
          pl.BlockSpec((_BR, H), lambda i: (i, 0)),
          pl.BlockSpec((_BR, 1), lambda i: (i, 0)),
          pl.BlockSpec((_BR, 1), lambda i: (i, 0)),
          pl.BlockSpec((H, H), lambda i: (0, 0)),
          pl.BlockSpec((H, H), lambda i: (0, 0)),
          pl.BlockSpec((1, H), lambda i: (0, 0)),
          pl.BlockSpec((1, H), lambda i: (0, 0)),
          pl.BlockSpec((1, H), lambda i: (0, 0)),
      ],
      out_specs=pl.BlockSpec((_BR, H), lambda i: (i, 0)),
      out_shape=jax.ShapeDtypeStruct((N, H), _f32),
  )(h, agg_a, agg_b, deg_a, deg_b, ws, wn, b, g, be)


def _pool_body(gid_ref, h_ref, wo_ref, bo_ref, o_ref, acc_ref, cnt_ref):
  i = pl.program_id(0)

  @pl.when(i == 0)
  def _():
    acc_ref[...] = jnp.zeros_like(acc_ref)
    cnt_ref[...] = jnp.zeros_like(cnt_ref)

  onehot = (gid_ref[...] ==
            lax.broadcasted_iota(jnp.int32, (1, G), 1)).astype(_f32)
  acc_ref[...] += lax.dot_general(onehot, h_ref[...], (((0,), (0,)), ((), ())),
                                  preferred_element_type=_f32)
  cnt_ref[...] += lax.dot_general(onehot, jnp.ones((_BR, 1), _f32),
                                  (((0,), (0,)), ((), ())),
                                  preferred_element_type=_f32)

  @pl.when(i == (N // _BR) - 1)
  def _():
    pooled = acc_ref[...] / jnp.maximum(cnt_ref[...], 1.0)
    o_ref[...] = jnp.dot(pooled, wo_ref[...],
                         preferred_element_type=_f32) + bo_ref[...]


def _pool(graph_ids2d, h, wo, bo):
  grid = N // _BR
  return pl.pallas_call(
      _pool_body,
      grid=(grid,),
      in_specs=[
          pl.BlockSpec((_BR, 1), lambda i: (i, 0)),
          pl.BlockSpec((_BR, H), lambda i: (i, 0)),
          pl.BlockSpec((H, OUT), lambda i: (0, 0)),
          pl.BlockSpec((1, OUT), lambda i: (0, 0)),
      ],
      out_specs=pl.BlockSpec((G, OUT), lambda i: (0, 0)),
      out_shape=jax.ShapeDtypeStruct((G, OUT), _f32),
      scratch_shapes=[
          pltpu.VMEM((G, H), _f32),
          pltpu.VMEM((G, 1), _f32),
      ],
  )(graph_ids2d, h, wo, bo)


def kernel(nodes, edge_index, graph_ids,
           W_in1, b_in1, W_in2, b_in2,
           W_self_0, W_neigh_0, bias_0, ln_g_0, ln_b_0,
           W_self_1, W_neigh_1, bias_1, ln_g_1, ln_b_1,
           W_self_2, W_neigh_2, bias_2, ln_g_2, ln_b_2,
           W_out, b_out):
  E = edge_index.shape[1]
  src = edge_index[0]
  dst = edge_index[1]
  src_p = jnp.concatenate(
      [src, jnp.zeros((EPAD - E,), jnp.int32)]).reshape(NW, CPW, CHUNK)
  dst_p = jnp.concatenate(
      [dst, jnp.full((EPAD - E,), N, jnp.int32)]).reshape(NW, CPW, CHUNK)

  z128 = jnp.zeros((CHUNK, H), _f32)

  ones_tab = jnp.ones((16, H), _f32)
  src_z = jnp.zeros_like(src_p)
  degp = _sc_agg(ones_tab, src_z, dst_p, z128)
  deg_a = lax.slice(degp, (0, 0), (N, 1))
  deg_b = lax.slice(degp, (NPAD, 0), (NPAD + N, 1))

  h = _mlp(nodes, W_in1, b_in1.reshape(1, H), W_in2, b_in2.reshape(1, H))

  layer_params = [
      (W_self_0, W_neigh_0, bias_0, ln_g_0, ln_b_0),
      (W_self_1, W_neigh_1, bias_1, ln_g_1, ln_b_1),
      (W_self_2, W_neigh_2, bias_2, ln_g_2, ln_b_2),
  ]
  for ws, wn, b, g, be in layer_params:
    aggp = _sc_agg(h, src_p, dst_p, z128)
    agg_a = lax.slice(aggp, (0, 0), (N, H))
    agg_b = lax.slice(aggp, (NPAD, 0), (NPAD + N, H))
    h = _layer(h, agg_a, agg_b, deg_a, deg_b,
               ws, wn, b.reshape(1, H), g.reshape(1, H), be.reshape(1, H))

  return _pool(graph_ids.reshape(N, 1), h, W_out, b_out.reshape(1, OUT))


# speedup vs baseline: 10.3289x; 10.3289x over previous
import functools

import jax
import jax.numpy as jnp
from jax import lax
from jax.experimental import pallas as pl
from jax.experimental.pallas import tpu as pltpu
from jax.experimental.pallas import tpu_sc as plsc

N = 10000
D = 128
H = 128
OUT = 16
G = 64

NC = 2
NS = 16
NW = NC * NS
CHUNK = 128
CPW = 79
EPAD = NW * CPW * CHUNK
NPAD = 10240
RPT = NPAD // NS
RCH = RPT // CHUNK

_f32 = jnp.float32


def _sc_mesh():
  return plsc.VectorSubcoreMesh(
      core_axis_name="c", subcore_axis_name="s",
      num_cores=NC, num_subcores=NS)


@functools.cache
def _sc_agg_kernel():
  @functools.partial(
      pl.kernel,
      out_type=jax.ShapeDtypeStruct((NC * NPAD, H), _f32),
      mesh=_sc_mesh(),
      scratch_types=[
          pltpu.VMEM_SHARED((NPAD, H), _f32),
          pltpu.VMEM((CPW, CHUNK), jnp.int32),
          pltpu.VMEM((CPW, CHUNK), jnp.int32),
          pltpu.VMEM((CHUNK, H), _f32),
          pltpu.SemaphoreType.DMA,
      ],
  )
  def body(h_hbm, src_hbm, dst_hbm, z_hbm, out_hbm,
           acc, src_v, dst_v, rows_v, sem):
    cid = lax.axis_index("c")
    sid = lax.axis_index("s")
    wid = sid * NC + cid
    pltpu.sync_copy(z_hbm, rows_v)
    for j in range(RCH):
      pltpu.sync_copy(rows_v, acc.at[pl.ds(sid * RPT + j * CHUNK, CHUNK)])
    pltpu.sync_copy(src_hbm.at[wid], src_v)
    pltpu.sync_copy(dst_hbm.at[wid], dst_v)
    plsc.subcore_barrier()

    def step(c, carry):
      pltpu.async_copy(h_hbm.at[src_v.at[c]], rows_v, sem).wait()
      pltpu.sync_copy(rows_v, acc.at[dst_v.at[c]], add=True)
      return carry

    lax.fori_loop(0, CPW, step, 0, unroll=False)
    plsc.subcore_barrier()
    for j in range(RCH):
      r0 = sid * RPT + j * CHUNK
      pltpu.sync_copy(acc.at[pl.ds(r0, CHUNK)], rows_v)
      pltpu.sync_copy(rows_v, out_hbm.at[pl.ds(cid * NPAD + r0, CHUNK)])

  return body


def _sc_agg(h, src_p, dst_p, z128):
  return _sc_agg_kernel()(h, src_p, dst_p, z128)


@functools.cache
def _sc_deg_kernel():
  @functools.partial(
      pl.kernel,
      out_type=jax.ShapeDtypeStruct((NC * NPAD, H), _f32),
      mesh=_sc_mesh(),
      scratch_types=[
          pltpu.VMEM_SHARED((NPAD, H), _f32),
          pltpu.VMEM((CPW, CHUNK), jnp.int32),
          pltpu.VMEM((CHUNK, H), _f32),
      ],
  )
  def body(dst_hbm, z_hbm, one_hbm, out_hbm, acc, dst_v, rows_v):
    cid = lax.axis_index("c")
    sid = lax.axis_index("s")
    wid = sid * NC + cid
    pltpu.sync_copy(z_hbm, rows_v)
    for j in range(RCH):
      pltpu.sync_copy(rows_v, acc.at[pl.ds(sid * RPT + j * CHUNK, CHUNK)])
    pltpu.sync_copy(one_hbm, rows_v)
    pltpu.sync_copy(dst_hbm.at[wid], dst_v)
    plsc.subcore_barrier()

    def step(c, carry):
      pltpu.sync_copy(rows_v, acc.at[dst_v.at[c]], add=True)
      return carry

    lax.fori_loop(0, CPW, step, 0, unroll=False)
    plsc.subcore_barrier()
    for j in range(RCH):
      r0 = sid * RPT + j * CHUNK
      pltpu.sync_copy(acc.at[pl.ds(r0, CHUNK)], rows_v)
      pltpu.sync_copy(rows_v, out_hbm.at[pl.ds(cid * NPAD + r0, CHUNK)])

  return body


def _sc_deg(dst_p, z128, one128):
  return _sc_deg_kernel()(dst_p, z128, one128)


_BR = 2000


def _mlp_body(x_ref, w1_ref, b1_ref, w2_ref, b2_ref, o_ref):
  x = x_ref[...]
  t = jnp.maximum(
      jnp.dot(x, w1_ref[...], preferred_element_type=_f32) + b1_ref[...], 0.0)
  o_ref[...] = jnp.dot(t, w2_ref[...], preferred_element_type=_f32) + b2_ref[...]


def _mlp(nodes, w1, b1, w2, b2):
  grid = N // _BR
  return pl.pallas_call(
      _mlp_body,
      grid=(grid,),
      in_specs=[
          pl.BlockSpec((_BR, D), lambda i: (i, 0)),
          pl.BlockSpec((D, H), lambda i: (0, 0)),
          pl.BlockSpec((1, H), lambda i: (0, 0)),
          pl.BlockSpec((H, H), lambda i: (0, 0)),
          pl.BlockSpec((1, H), lambda i: (0, 0)),
      ],
      out_specs=pl.BlockSpec((_BR, H), lambda i: (i, 0)),
      out_shape=jax.ShapeDtypeStruct((N, H), _f32),
  )(nodes, w1, b1, w2, b2)


def _layer_body(h_ref, aa_ref, ab_ref, da_ref, db_ref,
                ws_ref, wn_ref, b_ref, g_ref, be_ref, o_ref):
  deg = jnp.maximum(da_ref[...] + db_ref[...], 1.0)
  agg = (aa_ref[...] + ab_ref[...]) / deg
  r = (jnp.dot(h_ref[...], ws_ref[...], preferred_element_type=_f32)
       + jnp.dot(agg, wn_ref[...], preferred_element_type=_f32)
       + b_ref[...])
  r = jnp.maximum(r, 0.0)
  mu = jnp.mean(r, axis=-1, keepdims=True)
  var = jnp.mean((r - mu) ** 2, axis=-1, keepdims=True)
  o_ref[...] = (r - mu) * lax.rsqrt(var + 1e-5) * g_ref[...] + be_ref[...]


def _layer(h, agg_a, agg_b, deg_a, deg_b, ws, wn, b, g, be):
  grid = N // _BR
  return pl.pallas_call(
      _layer_body,
      grid=(grid,),
      in_specs=[
          pl.BlockSpec((_BR, H), lambda i: (i, 0)),
          pl.BlockSpec((_BR, H), lambda i: (i, 0)),
          pl.BlockSpec((_BR, H), lambda i: (i, 0)),
          pl.BlockSpec((_BR, 1), lambda i: (i, 0)),
          pl.BlockSpec((_BR, 1), lambda i: (i, 0)),
          pl.BlockSpec((H, H), lambda i: (0, 0)),
          pl.BlockSpec((H, H), lambda i: (0, 0)),
          pl.BlockSpec((1, H), lambda i: (0, 0)),
          pl.BlockSpec((1, H), lambda i: (0, 0)),
          pl.BlockSpec((1, H), lambda i: (0, 0)),
      ],
      out_specs=pl.BlockSpec((_BR, H), lambda i: (i, 0)),
      out_shape=jax.ShapeDtypeStruct((N, H), _f32),
  )(h, agg_a, agg_b, deg_a, deg_b, ws, wn, b, g, be)


def _pool_body(gid_ref, h_ref, wo_ref, bo_ref, o_ref, acc_ref, cnt_ref):
  i = pl.program_id(0)

  @pl.when(i == 0)
  def _():
    acc_ref[...] = jnp.zeros_like(acc_ref)
    cnt_ref[...] = jnp.zeros_like(cnt_ref)

  onehot = (gid_ref[...] ==
            lax.broadcasted_iota(jnp.int32, (1, G), 1)).astype(_f32)
  acc_ref[...] += lax.dot_general(onehot, h_ref[...], (((0,), (0,)), ((), ())),
                                  preferred_element_type=_f32)
  cnt_ref[...] += lax.dot_general(onehot, jnp.ones((_BR, 1), _f32),
                                  (((0,), (0,)), ((), ())),
                                  preferred_element_type=_f32)

  @pl.when(i == (N // _BR) - 1)
  def _():
    pooled = acc_ref[...] / jnp.maximum(cnt_ref[...], 1.0)
    o_ref[...] = jnp.dot(pooled, wo_ref[...],
                         preferred_element_type=_f32) + bo_ref[...]


def _pool(graph_ids2d, h, wo, bo):
  grid = N // _BR
  return pl.pallas_call(
      _pool_body,
      grid=(grid,),
      in_specs=[
          pl.BlockSpec((_BR, 1), lambda i: (i, 0)),
          pl.BlockSpec((_BR, H), lambda i: (i, 0)),
          pl.BlockSpec((H, OUT), lambda i: (0, 0)),
          pl.BlockSpec((1, OUT), lambda i: (0, 0)),
      ],
      out_specs=pl.BlockSpec((G, OUT), lambda i: (0, 0)),
      out_shape=jax.ShapeDtypeStruct((G, OUT), _f32),
      scratch_shapes=[
          pltpu.VMEM((G, H), _f32),
          pltpu.VMEM((G, 1), _f32),
      ],
  )(graph_ids2d, h, wo, bo)


def kernel(nodes, edge_index, graph_ids,
           W_in1, b_in1, W_in2, b_in2,
           W_self_0, W_neigh_0, bias_0, ln_g_0, ln_b_0,
           W_self_1, W_neigh_1, bias_1, ln_g_1, ln_b_1,
           W_self_2, W_neigh_2, bias_2, ln_g_2, ln_b_2,
           W_out, b_out):
  E = edge_index.shape[1]
  src = edge_index[0]
  dst = edge_index[1]
  src_p = jnp.concatenate(
      [src, jnp.zeros((EPAD - E,), jnp.int32)]).reshape(NW, CPW, CHUNK)
  dst_p = jnp.concatenate(
      [dst, jnp.full((EPAD - E,), N, jnp.int32)]).reshape(NW, CPW, CHUNK)

  z128 = jnp.zeros((CHUNK, H), _f32)

  one128 = jnp.ones((CHUNK, H), _f32)
  degp = _sc_deg(dst_p, z128, one128)
  deg_a = lax.slice(degp, (0, 0), (N, 1))
  deg_b = lax.slice(degp, (NPAD, 0), (NPAD + N, 1))

  h = _mlp(nodes, W_in1, b_in1.reshape(1, H), W_in2, b_in2.reshape(1, H))

  layer_params = [
      (W_self_0, W_neigh_0, bias_0, ln_g_0, ln_b_0),
      (W_self_1, W_neigh_1, bias_1, ln_g_1, ln_b_1),
      (W_self_2, W_neigh_2, bias_2, ln_g_2, ln_b_2),
  ]
  for ws, wn, b, g, be in layer_params:
    aggp = _sc_agg(h, src_p, dst_p, z128)
    agg_a = lax.slice(aggp, (0, 0), (N, H))
    agg_b = lax.slice(aggp, (NPAD, 0), (NPAD + N, H))
    h = _layer(h, agg_a, agg_b, deg_a, deg_b,
               ws, wn, b.reshape(1, H), g.reshape(1, H), be.reshape(1, H))

  return _pool(graph_ids.reshape(N, 1), h, W_out, b_out.reshape(1, OUT))
